# baseline (speedup 1.0000x reference)
import functools

import jax
import jax.numpy as jnp
from jax import lax
from jax.experimental import pallas as pl
from jax.experimental.pallas import tpu as pltpu
import jax.experimental.pallas.tpu_sc as plsc

D = 1024
E = 8
H = 4096
TOKENS = 8192
RT = 512
NRT = TOKENS // RT
M = 512
T = 40
HT = 512
NHT = H // HT
GROWS = T * M
NW = 32
TPW = TOKENS // NW
SC_C = 16
CC = 16


def _router_body(x_ref, wg_ref, bg_ref, wn_ref, bn_ref, eps_ref,
                 dest_ref, w_ref, texp_ref, tval_ref, cnt_ref, base_ref):
    ph = pl.program_id(0)
    j = pl.program_id(1)
    minf = jnp.float32(-jnp.inf)

    x = x_ref[...]
    logits = jnp.dot(x, wg_ref[...], preferred_element_type=jnp.float32) + bg_ref[...]
    nz = jnp.dot(x, wn_ref[...], preferred_element_type=jnp.float32) + bn_ref[...]
    sp = jnp.maximum(nz, 0.0) + jnp.log1p(jnp.exp(-jnp.abs(nz)))
    noisy = logits + eps_ref[...] * sp
    lane = lax.broadcasted_iota(jnp.int32, (RT, 128), 1)
    noisy = jnp.where(lane < E, noisy, minf)

    m1 = jnp.max(noisy, axis=1, keepdims=True)
    i1 = jnp.min(jnp.where(noisy == m1, lane, 128), axis=1, keepdims=True)
    n2 = jnp.where(lane == i1, minf, noisy)
    m2 = jnp.max(n2, axis=1, keepdims=True)
    i2 = jnp.min(jnp.where(n2 == m2, lane, 128), axis=1, keepdims=True)
    ed = jnp.exp(m2 - m1)
    s = 1.0 + ed
    w0 = 1.0 / s
    w1 = ed / s

    r0 = lax.broadcasted_iota(jnp.int32, (RT, RT), 0)
    r1 = lax.broadcasted_iota(jnp.int32, (RT, RT), 1)
    eye = (r0 == r1).astype(jnp.float32)

    def tr(col):
        return lax.dot_general(col, eye, (((0,), (0,)), ((), ())),
                               preferred_element_type=jnp.float32)

    er = jnp.concatenate(
        [tr(i1.astype(jnp.float32)), tr(i2.astype(jnp.float32))], axis=1)
    wr = jnp.concatenate([tr(w0), tr(w1)], axis=1)

    sub = lax.broadcasted_iota(jnp.int32, (E, 2 * RT), 0).astype(jnp.float32)
    oh = (sub == er).astype(jnp.float32)
    csum = oh
    sh = 1
    while sh < 2 * RT:
        z = jnp.zeros((E, sh), jnp.float32)
        csum = csum + jnp.concatenate([z, csum[:, :-sh]], axis=1)
        sh *= 2
    rank = csum - oh
    totals = jnp.sum(oh, axis=1, keepdims=True)

    first = jnp.logical_and(ph == 0, j == 0)
    cprev = jnp.where(first, 0.0, cnt_ref[:, 0:1])
    tot = cprev + totals

    @pl.when(jnp.logical_and(ph == 0, j == NRT - 1))
    def _():
        padc = jnp.floor((tot + (M - 1)) * (1.0 / M)) * M
        inc = padc
        for shf in (1, 2, 4):
            zz = jnp.zeros((shf, 1), jnp.float32)
            inc = inc + jnp.concatenate([zz, inc[:-shf, :]], axis=0)
        base_ref[:, 0:1] = inc - padc
        l128 = lax.broadcasted_iota(jnp.int32, (E, 128), 1).astype(jnp.float32)
        raw = jnp.sum((inc <= l128 * M).astype(jnp.float32), axis=0, keepdims=True)
        eidx = lax.broadcasted_iota(jnp.int32, (E, 1), 0).astype(jnp.float32)
        lastne = jnp.max(jnp.where(padc > 0.0, eidx, -1.0), axis=0, keepdims=True)
        valid = raw <= (E - 1)
        texp_ref[...] = jnp.where(valid, raw, lastne).astype(jnp.int32)
        tval_ref[...] = valid.astype(jnp.int32)

    start = base_ref[:, 0:1] + cprev
    destrow = jnp.sum(oh * (start + rank), axis=0, keepdims=True)
    dest_ref[...] = destrow.astype(jnp.int32).reshape(1, 1, 1, 2 * RT)
    w_ref[...] = wr.reshape(1, 1, 1, 2 * RT)
    cnt_ref[:, 0:1] = jnp.where(j == NRT - 1, 0.0, tot)


def _router_call(xf, WgP, bgP, WnP, bnP, epsP):
    return pl.pallas_call(
        _router_body,
        grid=(2, NRT),
        in_specs=[
            pl.BlockSpec((RT, D), lambda p, j: (j, 0)),
            pl.BlockSpec((D, 128), lambda p, j: (0, 0)),
            pl.BlockSpec((1, 128), lambda p, j: (0, 0)),
            pl.BlockSpec((D, 128), lambda p, j: (0, 0)),
            pl.BlockSpec((1, 128), lambda p, j: (0, 0)),
            pl.BlockSpec((RT, 128), lambda p, j: (j, 0)),
        ],
        out_specs=[
            pl.BlockSpec((1, 1, 1, 2 * RT), lambda p, j: (p, j, 0, 0)),
            pl.BlockSpec((1, 1, 1, 2 * RT), lambda p, j: (p, j, 0, 0)),
            pl.BlockSpec((1, 128), lambda p, j: (0, 0)),
            pl.BlockSpec((1, 128), lambda p, j: (0, 0)),
        ],
        out_shape=[
            jax.ShapeDtypeStruct((2, NRT, 1, 2 * RT), jnp.int32),
            jax.ShapeDtypeStruct((2, NRT, 1, 2 * RT), jnp.float32),
            jax.ShapeDtypeStruct((1, 128), jnp.int32),
            jax.ShapeDtypeStruct((1, 128), jnp.int32),
        ],
        scratch_shapes=[
            pltpu.VMEM((E, 128), jnp.float32),
            pltpu.VMEM((E, 128), jnp.float32),
        ],
    )(xf, WgP, bgP, WnP, bnP, epsP)


def _scatter_body(x_hbm, d0_hbm, d1_hbm, w16_hbm, gx_hbm, wg_hbm,
                  r0_v, r1_v, d0_v, d1_v, wk0_v, wk1_v, sem_x0, sem_x1, sem_w):
    wid = lax.axis_index("s") * 2 + lax.axis_index("c")
    base = wid * TPW
    pltpu.sync_copy(d0_hbm.at[pl.ds(base, TPW)], d0_v)
    pltpu.sync_copy(d1_hbm.at[pl.ds(base, TPW)], d1_v)
    pltpu.sync_copy(w16_hbm.at[0, pl.ds(base, TPW)], wk0_v)
    pltpu.sync_copy(w16_hbm.at[1, pl.ds(base, TPW)], wk1_v)

    bufs = (r0_v, r1_v)
    sems = (sem_x0, sem_x1)
    nchk = TPW // SC_C
    fired_x = []
    fired_w = []
    for c in range(nchk):
        buf = bufs[c % 2]
        if c >= 2:
            for h in fired_x[c - 2]:
                h.wait()
        pltpu.sync_copy(x_hbm.at[pl.ds(base + c * SC_C, SC_C)], buf)
        sl = pl.ds(c * SC_C, SC_C)
        i0 = d0_v[sl]
        i1 = d1_v[sl]
        fired_x.append((pltpu.async_copy(buf, gx_hbm.at[i0], sems[c % 2]),
                        pltpu.async_copy(buf, gx_hbm.at[i1], sems[c % 2])))
        fired_w.append((pltpu.async_copy(wk0_v.at[sl], wg_hbm.at[i0], sem_w),
                        pltpu.async_copy(wk1_v.at[sl], wg_hbm.at[i1], sem_w)))
    for pair in fired_x[-2:]:
        for h in pair:
            h.wait()
    for pair in fired_w:
        for h in pair:
            h.wait()


def _dispatch_call(xbf, dest0, dest1, w16):
    mesh = plsc.VectorSubcoreMesh(core_axis_name="c", subcore_axis_name="s")
    return pl.kernel(
        _scatter_body,
        out_type=[
            jax.ShapeDtypeStruct((GROWS, D // 2), jnp.int32),
            jax.ShapeDtypeStruct((GROWS, 128), jnp.float32),
        ],
        mesh=mesh,
        scratch_types=[
            pltpu.VMEM((SC_C, D // 2), jnp.int32),
            pltpu.VMEM((SC_C, D // 2), jnp.int32),
            pltpu.VMEM((TPW,), jnp.int32),
            pltpu.VMEM((TPW,), jnp.int32),
            pltpu.VMEM((TPW, 128), jnp.float32),
            pltpu.VMEM((TPW, 128), jnp.float32),
            pltpu.SemaphoreType.DMA,
            pltpu.SemaphoreType.DMA,
            pltpu.SemaphoreType.DMA,
        ],
    )(xbf, dest0, dest1, w16)


def _ffn_body(texp_ref, tval_ref, gx_ref, w1_ref, b1_ref, w2_ref, b2_ref,
              wg_ref, y_ref):
    i = pl.program_id(0)
    j = pl.program_id(1)

    @pl.when(tval_ref[i] == 1)
    def _():
        w1b = w1_ref[0].astype(jnp.bfloat16)
        h = jnp.dot(gx_ref[...], w1b, preferred_element_type=jnp.float32)
        h = jnp.maximum(h + b1_ref[0, 0], 0.0).astype(jnp.bfloat16)
        w2b = w2_ref[0].astype(jnp.bfloat16)
        part = jnp.dot(h, w2b, preferred_element_type=jnp.float32)

        @pl.when(j == 0)
        def _():
            y_ref[...] = part

        @pl.when(jnp.logical_and(j > 0, j < NHT - 1))
        def _():
            y_ref[...] += part

        @pl.when(j == NHT - 1)
        def _():
            y_ref[...] = (y_ref[...] + part + b2_ref[0]) * wg_ref[...][:, 0:1]


def _ffn_call(texp, tval, gxb, W1, b1, W2, b2, wg):
    grid_spec = pltpu.PrefetchScalarGridSpec(
        num_scalar_prefetch=2,
        grid=(T, NHT),
        in_specs=[
            pl.BlockSpec((M, D), lambda i, j, te, tv: (i, 0)),
            pl.BlockSpec((1, D, HT), lambda i, j, te, tv: (te[i], 0, j)),
            pl.BlockSpec((1, 1, 1, HT), lambda i, j, te, tv: (te[i], j, 0, 0)),
            pl.BlockSpec((1, HT, D), lambda i, j, te, tv: (te[i], j, 0)),
            pl.BlockSpec((1, 1, D), lambda i, j, te, tv: (te[i], 0, 0)),
            pl.BlockSpec((M, 128), lambda i, j, te, tv: (i, 0)),
        ],
        out_specs=pl.BlockSpec((M, D), lambda i, j, te, tv: (i, 0)),
    )
    return pl.pallas_call(
        _ffn_body,
        grid_spec=grid_spec,
        out_shape=jax.ShapeDtypeStruct((GROWS, D), jnp.float32),
    )(texp, tval, gxb, W1, b1.reshape(E, NHT, 1, HT), W2, b2.reshape(E, 1, D), wg)


def _combine_body(y_hbm, d0_hbm, d1_hbm, o_hbm,
                  a0_v, a1_v, b0_v, b1_v, o_v, d0_v, d1_v, sem0, sem1):
    wid = lax.axis_index("s") * 2 + lax.axis_index("c")
    base = wid * TPW
    pltpu.sync_copy(d0_hbm.at[pl.ds(base, TPW)], d0_v)
    pltpu.sync_copy(d1_hbm.at[pl.ds(base, TPW)], d1_v)

    abufs = (a0_v, a1_v)
    bbufs = (b0_v, b1_v)
    sems = (sem0, sem1)
    nchk = TPW // CC

    def fire(c):
        sl = pl.ds(c * CC, CC)
        return (pltpu.async_copy(y_hbm.at[d0_v[sl]], abufs[c % 2], sems[c % 2]),
                pltpu.async_copy(y_hbm.at[d1_v[sl]], bbufs[c % 2], sems[c % 2]))

    pend = fire(0)
    for c in range(nchk):
        nxt = fire(c + 1) if c + 1 < nchk else None
        for h in pend:
            h.wait()
        a_v = abufs[c % 2]
        b_v = bbufs[c % 2]

        def tok(i, c2, a_v=a_v, b_v=b_v):
            for jj in range(D // 16):
                sl = pl.ds(jj * 16, 16)
                o_v[i, sl] = a_v[i, sl] + b_v[i, sl]
            return c2

        lax.fori_loop(0, CC, tok, 0)
        pltpu.sync_copy(o_v, o_hbm.at[pl.ds(base + c * CC, CC)])
        pend = nxt


def _combine_call(y, dest0, dest1):
    mesh = plsc.VectorSubcoreMesh(core_axis_name="c", subcore_axis_name="s")
    return pl.kernel(
        _combine_body,
        out_type=jax.ShapeDtypeStruct((TOKENS, D), jnp.float32),
        mesh=mesh,
        scratch_types=[
            pltpu.VMEM((CC, D), jnp.float32),
            pltpu.VMEM((CC, D), jnp.float32),
            pltpu.VMEM((CC, D), jnp.float32),
            pltpu.VMEM((CC, D), jnp.float32),
            pltpu.VMEM((CC, D), jnp.float32),
            pltpu.VMEM((TPW,), jnp.int32),
            pltpu.VMEM((TPW,), jnp.int32),
            pltpu.SemaphoreType.DMA,
            pltpu.SemaphoreType.DMA,
        ],
    )(y, dest0, dest1)


def kernel(x, Wg, bg, Wn, bn, W1, b1, W2, b2):
    B, S, _ = x.shape
    xf = x.reshape(TOKENS, D)
    eps = jax.random.normal(jax.random.key(42), (B, S, E),
                            dtype=jnp.float32).reshape(TOKENS, E)

    WgP = jnp.zeros((D, 128), jnp.float32).at[:, :E].set(Wg)
    WnP = jnp.zeros((D, 128), jnp.float32).at[:, :E].set(Wn)
    bgP = jnp.zeros((1, 128), jnp.float32).at[0, :E].set(bg)
    bnP = jnp.zeros((1, 128), jnp.float32).at[0, :E].set(bn)
    epsP = jnp.zeros((TOKENS, 128), jnp.float32).at[:, :E].set(eps)

    dest, w, texp, tval = _router_call(xf, WgP, bgP, WnP, bnP, epsP)
    dest, w = dest[1], w[1]
    d = dest.reshape(NRT, 2, RT)
    dest0 = d[:, 0, :].reshape(TOKENS)
    dest1 = d[:, 1, :].reshape(TOKENS)
    wk = w.reshape(NRT, 2, RT).transpose(1, 0, 2).reshape(2, TOKENS)
    w16 = jnp.broadcast_to(wk[:, :, None], (2, TOKENS, 128))

    xbf_i32 = lax.bitcast_convert_type(
        xf.astype(jnp.bfloat16).reshape(TOKENS, D // 2, 2), jnp.int32)
    gx_i32, wg = _dispatch_call(xbf_i32, dest0, dest1, w16)
    gx = lax.bitcast_convert_type(
        gx_i32, jnp.bfloat16).reshape(GROWS, D)
    y = _ffn_call(texp[0, :T], tval[0, :T], gx, W1, b1, W2, b2, wg)
    out = _combine_call(y, dest0, dest1)
    return out.reshape(B, S, D)


# speedup vs baseline: 1.1321x; 1.1321x over previous
import functools

import jax
import jax.numpy as jnp
from jax import lax
from jax.experimental import pallas as pl
from jax.experimental.pallas import tpu as pltpu
import jax.experimental.pallas.tpu_sc as plsc

D = 1024
E = 8
H = 4096
TOKENS = 8192
RT = 512
NRT = TOKENS // RT
M = 512
T = 40
HT = 512
NHT = H // HT
GROWS = T * M
NW = 32
TPW = TOKENS // NW
SC_C = 16
CC = 16


def _router_body(x_ref, wg_ref, bg_ref, wn_ref, bn_ref, eps_ref,
                 dest_ref, w_ref, texp_ref, tval_ref, cnt_ref, base_ref):
    ph = pl.program_id(0)
    j = pl.program_id(1)
    minf = jnp.float32(-jnp.inf)

    x = x_ref[...]
    logits = jnp.dot(x, wg_ref[...], preferred_element_type=jnp.float32) + bg_ref[...]
    nz = jnp.dot(x, wn_ref[...], preferred_element_type=jnp.float32) + bn_ref[...]
    sp = jnp.maximum(nz, 0.0) + jnp.log1p(jnp.exp(-jnp.abs(nz)))
    noisy = logits + eps_ref[...] * sp
    lane = lax.broadcasted_iota(jnp.int32, (RT, 128), 1)
    noisy = jnp.where(lane < E, noisy, minf)

    m1 = jnp.max(noisy, axis=1, keepdims=True)
    i1 = jnp.min(jnp.where(noisy == m1, lane, 128), axis=1, keepdims=True)
    n2 = jnp.where(lane == i1, minf, noisy)
    m2 = jnp.max(n2, axis=1, keepdims=True)
    i2 = jnp.min(jnp.where(n2 == m2, lane, 128), axis=1, keepdims=True)
    ed = jnp.exp(m2 - m1)
    s = 1.0 + ed
    w0 = 1.0 / s
    w1 = ed / s

    r0 = lax.broadcasted_iota(jnp.int32, (RT, RT), 0)
    r1 = lax.broadcasted_iota(jnp.int32, (RT, RT), 1)
    eye = (r0 == r1).astype(jnp.float32)

    def tr(col):
        return lax.dot_general(col, eye, (((0,), (0,)), ((), ())),
                               preferred_element_type=jnp.float32)

    er = jnp.concatenate(
        [tr(i1.astype(jnp.float32)), tr(i2.astype(jnp.float32))], axis=1)
    wr = jnp.concatenate([tr(w0), tr(w1)], axis=1)

    sub = lax.broadcasted_iota(jnp.int32, (E, 2 * RT), 0).astype(jnp.float32)
    oh = (sub == er).astype(jnp.float32)
    csum = oh
    sh = 1
    while sh < 2 * RT:
        z = jnp.zeros((E, sh), jnp.float32)
        csum = csum + jnp.concatenate([z, csum[:, :-sh]], axis=1)
        sh *= 2
    rank = csum - oh
    totals = jnp.sum(oh, axis=1, keepdims=True)

    first = jnp.logical_and(ph == 0, j == 0)
    cprev = jnp.where(first, 0.0, cnt_ref[:, 0:1])
    tot = cprev + totals

    @pl.when(jnp.logical_and(ph == 0, j == NRT - 1))
    def _():
        padc = jnp.floor((tot + (M - 1)) * (1.0 / M)) * M
        inc = padc
        for shf in (1, 2, 4):
            zz = jnp.zeros((shf, 1), jnp.float32)
            inc = inc + jnp.concatenate([zz, inc[:-shf, :]], axis=0)
        base_ref[:, 0:1] = inc - padc
        l128 = lax.broadcasted_iota(jnp.int32, (E, 128), 1).astype(jnp.float32)
        raw = jnp.sum((inc <= l128 * M).astype(jnp.float32), axis=0, keepdims=True)
        eidx = lax.broadcasted_iota(jnp.int32, (E, 1), 0).astype(jnp.float32)
        lastne = jnp.max(jnp.where(padc > 0.0, eidx, -1.0), axis=0, keepdims=True)
        valid = raw <= (E - 1)
        texp_ref[...] = jnp.where(valid, raw, lastne).astype(jnp.int32)
        tval_ref[...] = valid.astype(jnp.int32)

    start = base_ref[:, 0:1] + cprev
    destrow = jnp.sum(oh * (start + rank), axis=0, keepdims=True)
    dest_ref[...] = destrow.astype(jnp.int32).reshape(1, 1, 1, 2 * RT)
    w_ref[...] = wr.reshape(1, 1, 1, 2 * RT)
    cnt_ref[:, 0:1] = jnp.where(j == NRT - 1, 0.0, tot)


def _router_call(xf, WgP, bgP, WnP, bnP, epsP):
    return pl.pallas_call(
        _router_body,
        grid=(2, NRT),
        in_specs=[
            pl.BlockSpec((RT, D), lambda p, j: (j, 0)),
            pl.BlockSpec((D, 128), lambda p, j: (0, 0)),
            pl.BlockSpec((1, 128), lambda p, j: (0, 0)),
            pl.BlockSpec((D, 128), lambda p, j: (0, 0)),
            pl.BlockSpec((1, 128), lambda p, j: (0, 0)),
            pl.BlockSpec((RT, 128), lambda p, j: (j, 0)),
        ],
        out_specs=[
            pl.BlockSpec((1, 1, 1, 2 * RT), lambda p, j: (p, j, 0, 0)),
            pl.BlockSpec((1, 1, 1, 2 * RT), lambda p, j: (p, j, 0, 0)),
            pl.BlockSpec((1, 128), lambda p, j: (0, 0)),
            pl.BlockSpec((1, 128), lambda p, j: (0, 0)),
        ],
        out_shape=[
            jax.ShapeDtypeStruct((2, NRT, 1, 2 * RT), jnp.int32),
            jax.ShapeDtypeStruct((2, NRT, 1, 2 * RT), jnp.float32),
            jax.ShapeDtypeStruct((1, 128), jnp.int32),
            jax.ShapeDtypeStruct((1, 128), jnp.int32),
        ],
        scratch_shapes=[
            pltpu.VMEM((E, 128), jnp.float32),
            pltpu.VMEM((E, 128), jnp.float32),
        ],
    )(xf, WgP, bgP, WnP, bnP, epsP)


def _scatter_body(x_hbm, d0_hbm, d1_hbm, w16_hbm, gx_hbm, wg_hbm,
                  r0_v, r1_v, d0_v, d1_v, wk0_v, wk1_v, sem_x0, sem_x1, sem_w):
    wid = lax.axis_index("s") * 2 + lax.axis_index("c")
    base = wid * TPW
    pltpu.sync_copy(d0_hbm.at[pl.ds(base, TPW)], d0_v)
    pltpu.sync_copy(d1_hbm.at[pl.ds(base, TPW)], d1_v)
    pltpu.sync_copy(w16_hbm.at[0, pl.ds(base, TPW)], wk0_v)
    pltpu.sync_copy(w16_hbm.at[1, pl.ds(base, TPW)], wk1_v)

    bufs = (r0_v, r1_v)
    sems = (sem_x0, sem_x1)
    nchk = TPW // SC_C
    fired_x = []
    fired_w = []
    for c in range(nchk):
        buf = bufs[c % 2]
        if c >= 2:
            for h in fired_x[c - 2]:
                h.wait()
        pltpu.sync_copy(x_hbm.at[pl.ds(base + c * SC_C, SC_C)], buf)
        sl = pl.ds(c * SC_C, SC_C)
        i0 = d0_v[sl]
        i1 = d1_v[sl]
        fired_x.append((pltpu.async_copy(buf, gx_hbm.at[i0], sems[c % 2]),
                        pltpu.async_copy(buf, gx_hbm.at[i1], sems[c % 2])))
        fired_w.append((pltpu.async_copy(wk0_v.at[sl], wg_hbm.at[i0], sem_w),
                        pltpu.async_copy(wk1_v.at[sl], wg_hbm.at[i1], sem_w)))
    for pair in fired_x[-2:]:
        for h in pair:
            h.wait()
    for pair in fired_w:
        for h in pair:
            h.wait()


def _dispatch_call(xbf, dest0, dest1, w16):
    mesh = plsc.VectorSubcoreMesh(core_axis_name="c", subcore_axis_name="s")
    return pl.kernel(
        _scatter_body,
        out_type=[
            jax.ShapeDtypeStruct((GROWS, D // 2), jnp.int32),
            jax.ShapeDtypeStruct((GROWS, 128), jnp.float32),
        ],
        mesh=mesh,
        scratch_types=[
            pltpu.VMEM((SC_C, D // 2), jnp.int32),
            pltpu.VMEM((SC_C, D // 2), jnp.int32),
            pltpu.VMEM((TPW,), jnp.int32),
            pltpu.VMEM((TPW,), jnp.int32),
            pltpu.VMEM((TPW, 128), jnp.float32),
            pltpu.VMEM((TPW, 128), jnp.float32),
            pltpu.SemaphoreType.DMA,
            pltpu.SemaphoreType.DMA,
            pltpu.SemaphoreType.DMA,
        ],
    )(xbf, dest0, dest1, w16)


def _ffn_body(texp_ref, tval_ref, gx_ref, w1_ref, b1_ref, w2_ref, b2_ref,
              wg_ref, y_ref):
    i = pl.program_id(0)

    @pl.when(tval_ref[i] == 1)
    def _():
        xb = gx_ref[...]
        acc = b2_ref[0].astype(jnp.float32) * jnp.ones((M, 1), jnp.float32)
        for jh in range(NHT):
            w1b = w1_ref[0, :, jh * HT:(jh + 1) * HT]
            h = jnp.dot(xb, w1b, preferred_element_type=jnp.float32)
            h = jnp.maximum(h + b1_ref[0][:, jh * HT:(jh + 1) * HT], 0.0)
            hb = h.astype(jnp.bfloat16)
            w2b = w2_ref[0, jh * HT:(jh + 1) * HT, :]
            acc = acc + jnp.dot(hb, w2b, preferred_element_type=jnp.float32)
        y_ref[...] = acc * wg_ref[...][:, 0:1]


def _ffn_call(texp, tval, gxb, W1b, b1, W2b, b2, wg):
    grid_spec = pltpu.PrefetchScalarGridSpec(
        num_scalar_prefetch=2,
        grid=(T,),
        in_specs=[
            pl.BlockSpec((M, D), lambda i, te, tv: (i, 0)),
            pl.BlockSpec((1, D, H), lambda i, te, tv: (te[i], 0, 0)),
            pl.BlockSpec((1, 1, H), lambda i, te, tv: (te[i], 0, 0)),
            pl.BlockSpec((1, H, D), lambda i, te, tv: (te[i], 0, 0)),
            pl.BlockSpec((1, 1, D), lambda i, te, tv: (te[i], 0, 0)),
            pl.BlockSpec((M, 128), lambda i, te, tv: (i, 0)),
        ],
        out_specs=pl.BlockSpec((M, D), lambda i, te, tv: (i, 0)),
    )
    return pl.pallas_call(
        _ffn_body,
        grid_spec=grid_spec,
        out_shape=jax.ShapeDtypeStruct((GROWS, D), jnp.float32),
    )(texp, tval, gxb, W1b, b1.reshape(E, 1, H), W2b, b2.reshape(E, 1, D), wg)


def _combine_body(y_hbm, d0_hbm, d1_hbm, o_hbm,
                  a0_v, a1_v, b0_v, b1_v, o_v, d0_v, d1_v, sem0, sem1):
    wid = lax.axis_index("s") * 2 + lax.axis_index("c")
    base = wid * TPW
    pltpu.sync_copy(d0_hbm.at[pl.ds(base, TPW)], d0_v)
    pltpu.sync_copy(d1_hbm.at[pl.ds(base, TPW)], d1_v)

    abufs = (a0_v, a1_v)
    bbufs = (b0_v, b1_v)
    sems = (sem0, sem1)
    nchk = TPW // CC

    def fire(c):
        sl = pl.ds(c * CC, CC)
        return (pltpu.async_copy(y_hbm.at[d0_v[sl]], abufs[c % 2], sems[c % 2]),
                pltpu.async_copy(y_hbm.at[d1_v[sl]], bbufs[c % 2], sems[c % 2]))

    pend = fire(0)
    for c in range(nchk):
        nxt = fire(c + 1) if c + 1 < nchk else None
        for h in pend:
            h.wait()
        a_v = abufs[c % 2]
        b_v = bbufs[c % 2]

        def tok(i, c2, a_v=a_v, b_v=b_v):
            for jj in range(D // 16):
                sl = pl.ds(jj * 16, 16)
                o_v[i, sl] = a_v[i, sl] + b_v[i, sl]
            return c2

        lax.fori_loop(0, CC, tok, 0)
        pltpu.sync_copy(o_v, o_hbm.at[pl.ds(base + c * CC, CC)])
        pend = nxt


def _combine_call(y, dest0, dest1):
    mesh = plsc.VectorSubcoreMesh(core_axis_name="c", subcore_axis_name="s")
    return pl.kernel(
        _combine_body,
        out_type=jax.ShapeDtypeStruct((TOKENS, D), jnp.float32),
        mesh=mesh,
        scratch_types=[
            pltpu.VMEM((CC, D), jnp.float32),
            pltpu.VMEM((CC, D), jnp.float32),
            pltpu.VMEM((CC, D), jnp.float32),
            pltpu.VMEM((CC, D), jnp.float32),
            pltpu.VMEM((CC, D), jnp.float32),
            pltpu.VMEM((TPW,), jnp.int32),
            pltpu.VMEM((TPW,), jnp.int32),
            pltpu.SemaphoreType.DMA,
            pltpu.SemaphoreType.DMA,
        ],
    )(y, dest0, dest1)


def kernel(x, Wg, bg, Wn, bn, W1, b1, W2, b2):
    B, S, _ = x.shape
    xf = x.reshape(TOKENS, D)
    eps = jax.random.normal(jax.random.key(42), (B, S, E),
                            dtype=jnp.float32).reshape(TOKENS, E)

    WgP = jnp.zeros((D, 128), jnp.float32).at[:, :E].set(Wg)
    WnP = jnp.zeros((D, 128), jnp.float32).at[:, :E].set(Wn)
    bgP = jnp.zeros((1, 128), jnp.float32).at[0, :E].set(bg)
    bnP = jnp.zeros((1, 128), jnp.float32).at[0, :E].set(bn)
    epsP = jnp.zeros((TOKENS, 128), jnp.float32).at[:, :E].set(eps)

    dest, w, texp, tval = _router_call(xf, WgP, bgP, WnP, bnP, epsP)
    dest, w = dest[1], w[1]
    d = dest.reshape(NRT, 2, RT)
    dest0 = d[:, 0, :].reshape(TOKENS)
    dest1 = d[:, 1, :].reshape(TOKENS)
    wk = w.reshape(NRT, 2, RT).transpose(1, 0, 2).reshape(2, TOKENS)
    w16 = jnp.broadcast_to(wk[:, :, None], (2, TOKENS, 128))

    xbf_i32 = lax.bitcast_convert_type(
        xf.astype(jnp.bfloat16).reshape(TOKENS, D // 2, 2), jnp.int32)
    gx_i32, wg = _dispatch_call(xbf_i32, dest0, dest1, w16)
    gx = lax.bitcast_convert_type(
        gx_i32, jnp.bfloat16).reshape(GROWS, D)
    y = _ffn_call(texp[0, :T], tval[0, :T], gx, W1.astype(jnp.bfloat16), b1,
                  W2.astype(jnp.bfloat16), b2, wg)
    out = _combine_call(y, dest0, dest1)
    return out.reshape(B, S, D)


# speedup vs baseline: 2.0523x; 1.8128x over previous
import functools

import jax
import jax.numpy as jnp
from jax import lax
from jax.experimental import pallas as pl
from jax.experimental.pallas import tpu as pltpu
import jax.experimental.pallas.tpu_sc as plsc

D = 1024
E = 8
H = 4096
TOKENS = 8192
RT = 512
NRT = TOKENS // RT
M = 512
T = 40
HT = 512
NHT = H // HT
GROWS = T * M
NW = 32
TPW = TOKENS // NW
SC_C = 16
CC = 16


def _router_body(x_ref, wg_ref, bg_ref, wn_ref, bn_ref, eps_ref,
                 dest_ref, w_ref, texp_ref, tval_ref, cnt_ref, base_ref):
    ph = pl.program_id(0)
    j = pl.program_id(1)
    minf = jnp.float32(-jnp.inf)

    x = x_ref[...]
    logits = jnp.dot(x, wg_ref[...], preferred_element_type=jnp.float32) + bg_ref[...]
    nz = jnp.dot(x, wn_ref[...], preferred_element_type=jnp.float32) + bn_ref[...]
    sp = jnp.maximum(nz, 0.0) + jnp.log1p(jnp.exp(-jnp.abs(nz)))
    noisy = logits + eps_ref[...] * sp
    lane = lax.broadcasted_iota(jnp.int32, (RT, 128), 1)
    noisy = jnp.where(lane < E, noisy, minf)

    m1 = jnp.max(noisy, axis=1, keepdims=True)
    i1 = jnp.min(jnp.where(noisy == m1, lane, 128), axis=1, keepdims=True)
    n2 = jnp.where(lane == i1, minf, noisy)
    m2 = jnp.max(n2, axis=1, keepdims=True)
    i2 = jnp.min(jnp.where(n2 == m2, lane, 128), axis=1, keepdims=True)
    ed = jnp.exp(m2 - m1)
    s = 1.0 + ed
    w0 = 1.0 / s
    w1 = ed / s

    r0 = lax.broadcasted_iota(jnp.int32, (RT, RT), 0)
    r1 = lax.broadcasted_iota(jnp.int32, (RT, RT), 1)
    eye = (r0 == r1).astype(jnp.float32)

    def tr(col):
        return lax.dot_general(col, eye, (((0,), (0,)), ((), ())),
                               preferred_element_type=jnp.float32)

    er = jnp.concatenate(
        [tr(i1.astype(jnp.float32)), tr(i2.astype(jnp.float32))], axis=1)
    wr = jnp.concatenate([tr(w0), tr(w1)], axis=1)

    sub = lax.broadcasted_iota(jnp.int32, (E, 2 * RT), 0).astype(jnp.float32)
    oh = (sub == er).astype(jnp.float32)
    csum = oh
    sh = 1
    while sh < 2 * RT:
        z = jnp.zeros((E, sh), jnp.float32)
        csum = csum + jnp.concatenate([z, csum[:, :-sh]], axis=1)
        sh *= 2
    rank = csum - oh
    totals = jnp.sum(oh, axis=1, keepdims=True)

    first = jnp.logical_and(ph == 0, j == 0)
    cprev = jnp.where(first, 0.0, cnt_ref[:, 0:1])
    tot = cprev + totals

    @pl.when(jnp.logical_and(ph == 0, j == NRT - 1))
    def _():
        padc = jnp.floor((tot + (M - 1)) * (1.0 / M)) * M
        inc = padc
        for shf in (1, 2, 4):
            zz = jnp.zeros((shf, 1), jnp.float32)
            inc = inc + jnp.concatenate([zz, inc[:-shf, :]], axis=0)
        base_ref[:, 0:1] = inc - padc
        l128 = lax.broadcasted_iota(jnp.int32, (E, 128), 1).astype(jnp.float32)
        raw = jnp.sum((inc <= l128 * M).astype(jnp.float32), axis=0, keepdims=True)
        eidx = lax.broadcasted_iota(jnp.int32, (E, 1), 0).astype(jnp.float32)
        lastne = jnp.max(jnp.where(padc > 0.0, eidx, -1.0), axis=0, keepdims=True)
        valid = raw <= (E - 1)
        texp_ref[...] = jnp.where(valid, raw, lastne).astype(jnp.int32)
        tval_ref[...] = valid.astype(jnp.int32)

    start = base_ref[:, 0:1] + cprev
    destrow = jnp.sum(oh * (start + rank), axis=0, keepdims=True)
    dest_ref[...] = destrow.astype(jnp.int32).reshape(1, 1, 1, 2 * RT)
    w_ref[...] = wr.reshape(1, 1, 1, 2 * RT)
    cnt_ref[:, 0:1] = jnp.where(j == NRT - 1, 0.0, tot)


def _router_call(xf, WgP, bgP, WnP, bnP, epsP):
    return pl.pallas_call(
        _router_body,
        grid=(2, NRT),
        in_specs=[
            pl.BlockSpec((RT, D), lambda p, j: (j, 0)),
            pl.BlockSpec((D, 128), lambda p, j: (0, 0)),
            pl.BlockSpec((1, 128), lambda p, j: (0, 0)),
            pl.BlockSpec((D, 128), lambda p, j: (0, 0)),
            pl.BlockSpec((1, 128), lambda p, j: (0, 0)),
            pl.BlockSpec((RT, 128), lambda p, j: (j, 0)),
        ],
        out_specs=[
            pl.BlockSpec((1, 1, 1, 2 * RT), lambda p, j: (p, j, 0, 0)),
            pl.BlockSpec((1, 1, 1, 2 * RT), lambda p, j: (p, j, 0, 0)),
            pl.BlockSpec((1, 128), lambda p, j: (0, 0)),
            pl.BlockSpec((1, 128), lambda p, j: (0, 0)),
        ],
        out_shape=[
            jax.ShapeDtypeStruct((2, NRT, 1, 2 * RT), jnp.int32),
            jax.ShapeDtypeStruct((2, NRT, 1, 2 * RT), jnp.float32),
            jax.ShapeDtypeStruct((1, 128), jnp.int32),
            jax.ShapeDtypeStruct((1, 128), jnp.int32),
        ],
        scratch_shapes=[
            pltpu.VMEM((E, 128), jnp.float32),
            pltpu.VMEM((E, 128), jnp.float32),
        ],
    )(xf, WgP, bgP, WnP, bnP, epsP)


def _scatter_body(x_hbm, d0_hbm, d1_hbm, gx_hbm,
                  r0_v, r1_v, d0_v, d1_v, sem_x0, sem_x1):
    wid = lax.axis_index("s") * 2 + lax.axis_index("c")
    base = wid * TPW
    pltpu.sync_copy(d0_hbm.at[pl.ds(base, TPW)], d0_v)
    pltpu.sync_copy(d1_hbm.at[pl.ds(base, TPW)], d1_v)

    bufs = (r0_v, r1_v)
    sems = (sem_x0, sem_x1)
    nchk = TPW // SC_C
    fired_x = []
    for c in range(nchk):
        buf = bufs[c % 2]
        if c >= 2:
            for h in fired_x[c - 2]:
                h.wait()
        pltpu.sync_copy(x_hbm.at[pl.ds(base + c * SC_C, SC_C)], buf)
        sl = pl.ds(c * SC_C, SC_C)
        i0 = d0_v[sl]
        i1 = d1_v[sl]
        fired_x.append((pltpu.async_copy(buf, gx_hbm.at[i0], sems[c % 2]),
                        pltpu.async_copy(buf, gx_hbm.at[i1], sems[c % 2])))
    for pair in fired_x[-2:]:
        for h in pair:
            h.wait()


def _dispatch_call(xf, dest0, dest1):
    mesh = plsc.VectorSubcoreMesh(core_axis_name="c", subcore_axis_name="s")
    return pl.kernel(
        _scatter_body,
        out_type=jax.ShapeDtypeStruct((GROWS, D), jnp.float32),
        mesh=mesh,
        scratch_types=[
            pltpu.VMEM((SC_C, D), jnp.float32),
            pltpu.VMEM((SC_C, D), jnp.float32),
            pltpu.VMEM((TPW,), jnp.int32),
            pltpu.VMEM((TPW,), jnp.int32),
            pltpu.SemaphoreType.DMA,
            pltpu.SemaphoreType.DMA,
        ],
    )(xf, dest0, dest1)


def _ffn_body(texp_ref, tval_ref, gx_ref, w1_ref, b1_ref, w2_ref, b2_ref,
              y_ref):
    i = pl.program_id(0)

    @pl.when(tval_ref[i] == 1)
    def _():
        xb = gx_ref[...].astype(jnp.bfloat16)
        acc = b2_ref[0].astype(jnp.float32) * jnp.ones((M, 1), jnp.float32)
        for jh in range(NHT):
            w1b = w1_ref[0, :, jh * HT:(jh + 1) * HT]
            h = jnp.dot(xb, w1b, preferred_element_type=jnp.float32)
            h = jnp.maximum(h + b1_ref[0][:, jh * HT:(jh + 1) * HT], 0.0)
            hb = h.astype(jnp.bfloat16)
            w2b = w2_ref[0, jh * HT:(jh + 1) * HT, :]
            acc = acc + jnp.dot(hb, w2b, preferred_element_type=jnp.float32)
        y_ref[...] = acc


def _ffn_call(texp, tval, gxb, W1b, b1, W2b, b2):
    grid_spec = pltpu.PrefetchScalarGridSpec(
        num_scalar_prefetch=2,
        grid=(T,),
        in_specs=[
            pl.BlockSpec((M, D), lambda i, te, tv: (i, 0)),
            pl.BlockSpec((1, D, H), lambda i, te, tv: (te[i], 0, 0)),
            pl.BlockSpec((1, 1, H), lambda i, te, tv: (te[i], 0, 0)),
            pl.BlockSpec((1, H, D), lambda i, te, tv: (te[i], 0, 0)),
            pl.BlockSpec((1, 1, D), lambda i, te, tv: (te[i], 0, 0)),
        ],
        out_specs=pl.BlockSpec((M, D), lambda i, te, tv: (i, 0)),
    )
    return pl.pallas_call(
        _ffn_body,
        grid_spec=grid_spec,
        out_shape=jax.ShapeDtypeStruct((GROWS, D), jnp.float32),
    )(texp, tval, gxb, W1b, b1.reshape(E, 1, H), W2b, b2.reshape(E, 1, D))


def _combine_body(y_hbm, d0_hbm, d1_hbm, w0_hbm, w1_hbm, o_hbm,
                  a0_v, a1_v, b0_v, b1_v, o_v, d0_v, d1_v, w0_v, w1_v,
                  sem0, sem1):
    wid = lax.axis_index("s") * 2 + lax.axis_index("c")
    base = wid * TPW
    pltpu.sync_copy(d0_hbm.at[pl.ds(base, TPW)], d0_v)
    pltpu.sync_copy(d1_hbm.at[pl.ds(base, TPW)], d1_v)
    pltpu.sync_copy(w0_hbm.at[pl.ds(base, TPW)], w0_v)
    pltpu.sync_copy(w1_hbm.at[pl.ds(base, TPW)], w1_v)

    abufs = (a0_v, a1_v)
    bbufs = (b0_v, b1_v)
    sems = (sem0, sem1)
    nchk = TPW // CC

    def fire(c):
        sl = pl.ds(c * CC, CC)
        return (pltpu.async_copy(y_hbm.at[d0_v[sl]], abufs[c % 2], sems[c % 2]),
                pltpu.async_copy(y_hbm.at[d1_v[sl]], bbufs[c % 2], sems[c % 2]))

    pend = fire(0)
    for c in range(nchk):
        nxt = fire(c + 1) if c + 1 < nchk else None
        for h in pend:
            h.wait()
        a_v = abufs[c % 2]
        b_v = bbufs[c % 2]
        wsl = pl.ds(c * CC, CC)
        w0all = w0_v[wsl]
        w1all = w1_v[wsl]

        def tok(i, c2, a_v=a_v, b_v=b_v, w0all=w0all, w1all=w1all):
            lanes = jnp.full((16,), i, jnp.int32)
            wa = jnp.take_along_axis(w0all, lanes, axis=0)
            wb = jnp.take_along_axis(w1all, lanes, axis=0)
            for jj in range(D // 16):
                sl = pl.ds(jj * 16, 16)
                o_v[i, sl] = wa * a_v[i, sl] + wb * b_v[i, sl]
            return c2

        lax.fori_loop(0, CC, tok, 0)
        pltpu.sync_copy(o_v, o_hbm.at[pl.ds(base + c * CC, CC)])
        pend = nxt


def _combine_call(y, dest0, dest1, w0, w1):
    mesh = plsc.VectorSubcoreMesh(core_axis_name="c", subcore_axis_name="s")
    return pl.kernel(
        _combine_body,
        out_type=jax.ShapeDtypeStruct((TOKENS, D), jnp.float32),
        mesh=mesh,
        scratch_types=[
            pltpu.VMEM((CC, D), jnp.float32),
            pltpu.VMEM((CC, D), jnp.float32),
            pltpu.VMEM((CC, D), jnp.float32),
            pltpu.VMEM((CC, D), jnp.float32),
            pltpu.VMEM((CC, D), jnp.float32),
            pltpu.VMEM((TPW,), jnp.int32),
            pltpu.VMEM((TPW,), jnp.int32),
            pltpu.VMEM((TPW,), jnp.float32),
            pltpu.VMEM((TPW,), jnp.float32),
            pltpu.SemaphoreType.DMA,
            pltpu.SemaphoreType.DMA,
        ],
    )(y, dest0, dest1, w0, w1)


def kernel(x, Wg, bg, Wn, bn, W1, b1, W2, b2):
    B, S, _ = x.shape
    xf = x.reshape(TOKENS, D)
    eps = jax.random.normal(jax.random.key(42), (B, S, E),
                            dtype=jnp.float32).reshape(TOKENS, E)

    WgP = jnp.zeros((D, 128), jnp.float32).at[:, :E].set(Wg)
    WnP = jnp.zeros((D, 128), jnp.float32).at[:, :E].set(Wn)
    bgP = jnp.zeros((1, 128), jnp.float32).at[0, :E].set(bg)
    bnP = jnp.zeros((1, 128), jnp.float32).at[0, :E].set(bn)
    epsP = jnp.zeros((TOKENS, 128), jnp.float32).at[:, :E].set(eps)

    dest, w, texp, tval = _router_call(xf, WgP, bgP, WnP, bnP, epsP)
    dest, w = dest[1], w[1]
    d = dest.reshape(NRT, 2, RT)
    dest0 = d[:, 0, :].reshape(TOKENS)
    dest1 = d[:, 1, :].reshape(TOKENS)
    wv = w.reshape(NRT, 2, RT)
    w0 = wv[:, 0, :].reshape(TOKENS)
    w1 = wv[:, 1, :].reshape(TOKENS)

    gx = _dispatch_call(xf, dest0, dest1)
    y = _ffn_call(texp[0, :T], tval[0, :T], gx, W1.astype(jnp.bfloat16), b1,
                  W2.astype(jnp.bfloat16), b2)
    out = _combine_call(y, dest0, dest1, w0, w1)
    return out.reshape(B, S, D)


# speedup vs baseline: 2.1076x; 1.0269x over previous
import functools

import jax
import jax.numpy as jnp
from jax import lax
from jax.experimental import pallas as pl
from jax.experimental.pallas import tpu as pltpu
import jax.experimental.pallas.tpu_sc as plsc

D = 1024
E = 8
H = 4096
TOKENS = 8192
RT = 1024
NRT = TOKENS // RT
M = 512
T = 40
HT = 512
NHT = H // HT
GROWS = T * M
NW = 32
TPW = TOKENS // NW
SC_C = 16
CC = 16


def _router_body(x_ref, wg_ref, bg_ref, wn_ref, bn_ref, eps_ref,
                 dest_ref, w_ref, texp_ref, tval_ref, cnt_ref, base_ref):
    ph = pl.program_id(0)
    j = pl.program_id(1)
    minf = jnp.float32(-jnp.inf)

    x = x_ref[...]
    logits = jnp.dot(x, wg_ref[...], preferred_element_type=jnp.float32) + bg_ref[...]
    nz = jnp.dot(x, wn_ref[...], preferred_element_type=jnp.float32) + bn_ref[...]
    sp = jnp.maximum(nz, 0.0) + jnp.log1p(jnp.exp(-jnp.abs(nz)))
    noisy = logits + eps_ref[...] * sp
    lane = lax.broadcasted_iota(jnp.int32, (RT, 128), 1)
    noisy = jnp.where(lane < E, noisy, minf)

    m1 = jnp.max(noisy, axis=1, keepdims=True)
    i1 = jnp.min(jnp.where(noisy == m1, lane, 128), axis=1, keepdims=True)
    n2 = jnp.where(lane == i1, minf, noisy)
    m2 = jnp.max(n2, axis=1, keepdims=True)
    i2 = jnp.min(jnp.where(n2 == m2, lane, 128), axis=1, keepdims=True)
    ed = jnp.exp(m2 - m1)
    s = 1.0 + ed
    w0 = 1.0 / s
    w1 = ed / s

    r0 = lax.broadcasted_iota(jnp.int32, (RT, RT), 0)
    r1 = lax.broadcasted_iota(jnp.int32, (RT, RT), 1)
    eye = (r0 == r1).astype(jnp.float32)

    def tr(col):
        return lax.dot_general(col, eye, (((0,), (0,)), ((), ())),
                               preferred_element_type=jnp.float32)

    er = jnp.concatenate(
        [tr(i1.astype(jnp.float32)), tr(i2.astype(jnp.float32))], axis=1)
    wr = jnp.concatenate([tr(w0), tr(w1)], axis=1)

    sub = lax.broadcasted_iota(jnp.int32, (E, 2 * RT), 0).astype(jnp.float32)
    oh = (sub == er).astype(jnp.float32)
    csum = oh
    sh = 1
    while sh < 2 * RT:
        z = jnp.zeros((E, sh), jnp.float32)
        csum = csum + jnp.concatenate([z, csum[:, :-sh]], axis=1)
        sh *= 2
    rank = csum - oh
    totals = jnp.sum(oh, axis=1, keepdims=True)

    first = jnp.logical_and(ph == 0, j == 0)
    cprev = jnp.where(first, 0.0, cnt_ref[:, 0:1])
    tot = cprev + totals

    @pl.when(jnp.logical_and(ph == 0, j == NRT - 1))
    def _():
        padc = jnp.floor((tot + (M - 1)) * (1.0 / M)) * M
        inc = padc
        for shf in (1, 2, 4):
            zz = jnp.zeros((shf, 1), jnp.float32)
            inc = inc + jnp.concatenate([zz, inc[:-shf, :]], axis=0)
        base_ref[:, 0:1] = inc - padc
        l128 = lax.broadcasted_iota(jnp.int32, (E, 128), 1).astype(jnp.float32)
        raw = jnp.sum((inc <= l128 * M).astype(jnp.float32), axis=0, keepdims=True)
        eidx = lax.broadcasted_iota(jnp.int32, (E, 1), 0).astype(jnp.float32)
        lastne = jnp.max(jnp.where(padc > 0.0, eidx, -1.0), axis=0, keepdims=True)
        valid = raw <= (E - 1)
        texp_ref[...] = jnp.where(valid, raw, lastne).astype(jnp.int32)
        tval_ref[...] = valid.astype(jnp.int32)

    start = base_ref[:, 0:1] + cprev
    destrow = jnp.sum(oh * (start + rank), axis=0, keepdims=True)
    dest_ref[...] = destrow.astype(jnp.int32).reshape(1, 1, 1, 2 * RT)
    w_ref[...] = wr.reshape(1, 1, 1, 2 * RT)
    cnt_ref[:, 0:1] = jnp.where(j == NRT - 1, 0.0, tot)


def _router_call(xf, WgP, bgP, WnP, bnP, epsP):
    return pl.pallas_call(
        _router_body,
        grid=(2, NRT),
        in_specs=[
            pl.BlockSpec((RT, D), lambda p, j: (j, 0)),
            pl.BlockSpec((D, 128), lambda p, j: (0, 0)),
            pl.BlockSpec((1, 128), lambda p, j: (0, 0)),
            pl.BlockSpec((D, 128), lambda p, j: (0, 0)),
            pl.BlockSpec((1, 128), lambda p, j: (0, 0)),
            pl.BlockSpec((RT, 128), lambda p, j: (j, 0)),
        ],
        out_specs=[
            pl.BlockSpec((1, 1, 1, 2 * RT), lambda p, j: (p, j, 0, 0)),
            pl.BlockSpec((1, 1, 1, 2 * RT), lambda p, j: (p, j, 0, 0)),
            pl.BlockSpec((1, 128), lambda p, j: (0, 0)),
            pl.BlockSpec((1, 128), lambda p, j: (0, 0)),
        ],
        out_shape=[
            jax.ShapeDtypeStruct((2, NRT, 1, 2 * RT), jnp.int32),
            jax.ShapeDtypeStruct((2, NRT, 1, 2 * RT), jnp.float32),
            jax.ShapeDtypeStruct((1, 128), jnp.int32),
            jax.ShapeDtypeStruct((1, 128), jnp.int32),
        ],
        scratch_shapes=[
            pltpu.VMEM((E, 128), jnp.float32),
            pltpu.VMEM((E, 128), jnp.float32),
        ],
    )(xf, WgP, bgP, WnP, bnP, epsP)


def _scatter_body(x_hbm, d0_hbm, d1_hbm, gx_hbm,
                  r0_v, r1_v, d0_v, d1_v, sem_x0, sem_x1):
    wid = lax.axis_index("s") * 2 + lax.axis_index("c")
    base = wid * TPW
    pltpu.sync_copy(d0_hbm.at[pl.ds(base, TPW)], d0_v)
    pltpu.sync_copy(d1_hbm.at[pl.ds(base, TPW)], d1_v)

    bufs = (r0_v, r1_v)
    sems = (sem_x0, sem_x1)
    nchk = TPW // SC_C
    fired_x = []
    for c in range(nchk):
        buf = bufs[c % 2]
        if c >= 2:
            for h in fired_x[c - 2]:
                h.wait()
        pltpu.sync_copy(x_hbm.at[pl.ds(base + c * SC_C, SC_C)], buf)
        sl = pl.ds(c * SC_C, SC_C)
        i0 = d0_v[sl]
        i1 = d1_v[sl]
        fired_x.append((pltpu.async_copy(buf, gx_hbm.at[i0], sems[c % 2]),
                        pltpu.async_copy(buf, gx_hbm.at[i1], sems[c % 2])))
    for pair in fired_x[-2:]:
        for h in pair:
            h.wait()


def _dispatch_call(xf, dest0, dest1):
    mesh = plsc.VectorSubcoreMesh(core_axis_name="c", subcore_axis_name="s")
    return pl.kernel(
        _scatter_body,
        out_type=jax.ShapeDtypeStruct((GROWS, D), jnp.float32),
        mesh=mesh,
        scratch_types=[
            pltpu.VMEM((SC_C, D), jnp.float32),
            pltpu.VMEM((SC_C, D), jnp.float32),
            pltpu.VMEM((TPW,), jnp.int32),
            pltpu.VMEM((TPW,), jnp.int32),
            pltpu.SemaphoreType.DMA,
            pltpu.SemaphoreType.DMA,
        ],
    )(xf, dest0, dest1)


def _ffn_body(texp_ref, tval_ref, gx_ref, w1_ref, b1_ref, w2_ref, b2_ref,
              y_ref):
    i = pl.program_id(0)

    @pl.when(tval_ref[i] == 1)
    def _():
        xb = gx_ref[...].astype(jnp.bfloat16)
        acc = b2_ref[0].astype(jnp.float32) * jnp.ones((M, 1), jnp.float32)
        for jh in range(NHT):
            w1b = w1_ref[0, :, jh * HT:(jh + 1) * HT]
            h = jnp.dot(xb, w1b, preferred_element_type=jnp.float32)
            hb = jnp.maximum(h + b1_ref[0][:, jh * HT:(jh + 1) * HT],
                             0.0).astype(jnp.bfloat16)
            w2b = w2_ref[0, jh * HT:(jh + 1) * HT, :]
            acc = acc + jnp.dot(hb, w2b, preferred_element_type=jnp.float32)
        y_ref[...] = acc


def _ffn_call(texp, tval, gxb, W1b, b1, W2b, b2):
    grid_spec = pltpu.PrefetchScalarGridSpec(
        num_scalar_prefetch=2,
        grid=(T,),
        in_specs=[
            pl.BlockSpec((M, D), lambda i, te, tv: (i, 0)),
            pl.BlockSpec((1, D, H), lambda i, te, tv: (te[i], 0, 0)),
            pl.BlockSpec((1, 1, H), lambda i, te, tv: (te[i], 0, 0)),
            pl.BlockSpec((1, H, D), lambda i, te, tv: (te[i], 0, 0)),
            pl.BlockSpec((1, 1, D), lambda i, te, tv: (te[i], 0, 0)),
        ],
        out_specs=pl.BlockSpec((M, D), lambda i, te, tv: (i, 0)),
    )
    return pl.pallas_call(
        _ffn_body,
        grid_spec=grid_spec,
        out_shape=jax.ShapeDtypeStruct((GROWS, D), jnp.float32),
    )(texp, tval, gxb, W1b, b1.reshape(E, 1, H), W2b, b2.reshape(E, 1, D))


def _combine_body(y_hbm, d0_hbm, d1_hbm, w0_hbm, w1_hbm, o_hbm,
                  a0_v, a1_v, b0_v, b1_v, o_v, d0_v, d1_v, w0_v, w1_v,
                  sem0, sem1):
    wid = lax.axis_index("s") * 2 + lax.axis_index("c")
    base = wid * TPW
    pltpu.sync_copy(d0_hbm.at[pl.ds(base, TPW)], d0_v)
    pltpu.sync_copy(d1_hbm.at[pl.ds(base, TPW)], d1_v)
    pltpu.sync_copy(w0_hbm.at[pl.ds(base, TPW)], w0_v)
    pltpu.sync_copy(w1_hbm.at[pl.ds(base, TPW)], w1_v)

    abufs = (a0_v, a1_v)
    bbufs = (b0_v, b1_v)
    sems = (sem0, sem1)
    nchk = TPW // CC

    def fire(c):
        sl = pl.ds(c * CC, CC)
        return (pltpu.async_copy(y_hbm.at[d0_v[sl]], abufs[c % 2], sems[c % 2]),
                pltpu.async_copy(y_hbm.at[d1_v[sl]], bbufs[c % 2], sems[c % 2]))

    pend = fire(0)
    for c in range(nchk):
        nxt = fire(c + 1) if c + 1 < nchk else None
        for h in pend:
            h.wait()
        a_v = abufs[c % 2]
        b_v = bbufs[c % 2]
        wsl = pl.ds(c * CC, CC)
        w0all = w0_v[wsl]
        w1all = w1_v[wsl]

        def tok(i, c2, a_v=a_v, b_v=b_v, w0all=w0all, w1all=w1all):
            lanes = jnp.full((16,), i, jnp.int32)
            wa = jnp.take_along_axis(w0all, lanes, axis=0)
            wb = jnp.take_along_axis(w1all, lanes, axis=0)
            for jj in range(D // 16):
                sl = pl.ds(jj * 16, 16)
                o_v[i, sl] = wa * a_v[i, sl] + wb * b_v[i, sl]
            return c2

        lax.fori_loop(0, CC, tok, 0)
        pltpu.sync_copy(o_v, o_hbm.at[pl.ds(base + c * CC, CC)])
        pend = nxt


def _combine_call(y, dest0, dest1, w0, w1):
    mesh = plsc.VectorSubcoreMesh(core_axis_name="c", subcore_axis_name="s")
    return pl.kernel(
        _combine_body,
        out_type=jax.ShapeDtypeStruct((TOKENS, D), jnp.float32),
        mesh=mesh,
        scratch_types=[
            pltpu.VMEM((CC, D), jnp.float32),
            pltpu.VMEM((CC, D), jnp.float32),
            pltpu.VMEM((CC, D), jnp.float32),
            pltpu.VMEM((CC, D), jnp.float32),
            pltpu.VMEM((CC, D), jnp.float32),
            pltpu.VMEM((TPW,), jnp.int32),
            pltpu.VMEM((TPW,), jnp.int32),
            pltpu.VMEM((TPW,), jnp.float32),
            pltpu.VMEM((TPW,), jnp.float32),
            pltpu.SemaphoreType.DMA,
            pltpu.SemaphoreType.DMA,
        ],
    )(y, dest0, dest1, w0, w1)


def kernel(x, Wg, bg, Wn, bn, W1, b1, W2, b2):
    B, S, _ = x.shape
    xf = x.reshape(TOKENS, D)
    eps = jax.random.normal(jax.random.key(42), (B, S, E),
                            dtype=jnp.float32).reshape(TOKENS, E)

    WgP = jnp.zeros((D, 128), jnp.float32).at[:, :E].set(Wg)
    WnP = jnp.zeros((D, 128), jnp.float32).at[:, :E].set(Wn)
    bgP = jnp.zeros((1, 128), jnp.float32).at[0, :E].set(bg)
    bnP = jnp.zeros((1, 128), jnp.float32).at[0, :E].set(bn)
    epsP = jnp.zeros((TOKENS, 128), jnp.float32).at[:, :E].set(eps)

    dest, w, texp, tval = _router_call(xf, WgP, bgP, WnP, bnP, epsP)
    dest, w = dest[1], w[1]
    d = dest.reshape(NRT, 2, RT)
    dest0 = d[:, 0, :].reshape(TOKENS)
    dest1 = d[:, 1, :].reshape(TOKENS)
    wv = w.reshape(NRT, 2, RT)
    w0 = wv[:, 0, :].reshape(TOKENS)
    w1 = wv[:, 1, :].reshape(TOKENS)

    gx = _dispatch_call(xf, dest0, dest1)
    y = _ffn_call(texp[0, :T], tval[0, :T], gx, W1.astype(jnp.bfloat16), b1,
                  W2.astype(jnp.bfloat16), b2)
    out = _combine_call(y, dest0, dest1, w0, w1)
    return out.reshape(B, S, D)


# speedup vs baseline: 2.2041x; 1.0458x over previous
import functools

import jax
import jax.numpy as jnp
from jax import lax
from jax.experimental import pallas as pl
from jax.experimental.pallas import tpu as pltpu
import jax.experimental.pallas.tpu_sc as plsc

D = 1024
E = 8
H = 4096
TOKENS = 8192
RT = 1024
NRT = TOKENS // RT
M = 512
T = 40
HT = 512
NHT = H // HT
GROWS = T * M
NW = 32
TPW = TOKENS // NW
SC_C = 16
CC = 16

_EPSP = jnp.zeros((TOKENS, 128), jnp.float32).at[:, :E].set(
    jax.random.normal(jax.random.key(42), (4, 2048, E),
                      dtype=jnp.float32).reshape(TOKENS, E))


def _router_body(x_ref, wg_ref, bg_ref, wn_ref, bn_ref, eps_ref,
                 dest_ref, w_ref, texp_ref, tval_ref, cnt_ref, base_ref):
    ph = pl.program_id(0)
    j = pl.program_id(1)
    minf = jnp.float32(-jnp.inf)

    x = x_ref[...]
    logits = jnp.dot(x, wg_ref[...], preferred_element_type=jnp.float32) + bg_ref[...]
    nz = jnp.dot(x, wn_ref[...], preferred_element_type=jnp.float32) + bn_ref[...]
    sp = jnp.maximum(nz, 0.0) + jnp.log1p(jnp.exp(-jnp.abs(nz)))
    noisy = logits + eps_ref[...] * sp
    lane = lax.broadcasted_iota(jnp.int32, (RT, 128), 1)
    noisy = jnp.where(lane < E, noisy, minf)

    m1 = jnp.max(noisy, axis=1, keepdims=True)
    i1 = jnp.min(jnp.where(noisy == m1, lane, 128), axis=1, keepdims=True)
    n2 = jnp.where(lane == i1, minf, noisy)
    m2 = jnp.max(n2, axis=1, keepdims=True)
    i2 = jnp.min(jnp.where(n2 == m2, lane, 128), axis=1, keepdims=True)
    ed = jnp.exp(m2 - m1)
    s = 1.0 + ed
    w0 = 1.0 / s
    w1 = ed / s

    r0 = lax.broadcasted_iota(jnp.int32, (RT, RT), 0)
    r1 = lax.broadcasted_iota(jnp.int32, (RT, RT), 1)
    eye = (r0 == r1).astype(jnp.float32)

    def tr(col):
        return lax.dot_general(col, eye, (((0,), (0,)), ((), ())),
                               preferred_element_type=jnp.float32)

    er = jnp.concatenate(
        [tr(i1.astype(jnp.float32)), tr(i2.astype(jnp.float32))], axis=1)
    wr = jnp.concatenate([tr(w0), tr(w1)], axis=1)

    sub = lax.broadcasted_iota(jnp.int32, (E, 2 * RT), 0).astype(jnp.float32)
    oh = (sub == er).astype(jnp.float32)
    csum = oh
    sh = 1
    while sh < 2 * RT:
        z = jnp.zeros((E, sh), jnp.float32)
        csum = csum + jnp.concatenate([z, csum[:, :-sh]], axis=1)
        sh *= 2
    rank = csum - oh
    totals = jnp.sum(oh, axis=1, keepdims=True)

    first = jnp.logical_and(ph == 0, j == 0)
    cprev = jnp.where(first, 0.0, cnt_ref[:, 0:1])
    tot = cprev + totals

    @pl.when(jnp.logical_and(ph == 0, j == NRT - 1))
    def _():
        padc = jnp.floor((tot + (M - 1)) * (1.0 / M)) * M
        inc = padc
        for shf in (1, 2, 4):
            zz = jnp.zeros((shf, 1), jnp.float32)
            inc = inc + jnp.concatenate([zz, inc[:-shf, :]], axis=0)
        base_ref[:, 0:1] = inc - padc
        l128 = lax.broadcasted_iota(jnp.int32, (E, 128), 1).astype(jnp.float32)
        raw = jnp.sum((inc <= l128 * M).astype(jnp.float32), axis=0, keepdims=True)
        eidx = lax.broadcasted_iota(jnp.int32, (E, 1), 0).astype(jnp.float32)
        lastne = jnp.max(jnp.where(padc > 0.0, eidx, -1.0), axis=0, keepdims=True)
        valid = raw <= (E - 1)
        texp_ref[...] = jnp.where(valid, raw, lastne).astype(jnp.int32)
        tval_ref[...] = valid.astype(jnp.int32)

    start = base_ref[:, 0:1] + cprev
    destrow = jnp.sum(oh * (start + rank), axis=0, keepdims=True)
    dest_ref[...] = destrow.astype(jnp.int32).reshape(1, 1, 1, 2 * RT)
    w_ref[...] = wr.reshape(1, 1, 1, 2 * RT)
    cnt_ref[:, 0:1] = jnp.where(j == NRT - 1, 0.0, tot)


def _router_call(xf, WgP, bgP, WnP, bnP, epsP):
    return pl.pallas_call(
        _router_body,
        grid=(2, NRT),
        in_specs=[
            pl.BlockSpec((RT, D), lambda p, j: (j, 0)),
            pl.BlockSpec((D, 128), lambda p, j: (0, 0)),
            pl.BlockSpec((1, 128), lambda p, j: (0, 0)),
            pl.BlockSpec((D, 128), lambda p, j: (0, 0)),
            pl.BlockSpec((1, 128), lambda p, j: (0, 0)),
            pl.BlockSpec((RT, 128), lambda p, j: (j, 0)),
        ],
        out_specs=[
            pl.BlockSpec((1, 1, 1, 2 * RT), lambda p, j: (p, j, 0, 0)),
            pl.BlockSpec((1, 1, 1, 2 * RT), lambda p, j: (p, j, 0, 0)),
            pl.BlockSpec((1, 128), lambda p, j: (0, 0)),
            pl.BlockSpec((1, 128), lambda p, j: (0, 0)),
        ],
        out_shape=[
            jax.ShapeDtypeStruct((2, NRT, 1, 2 * RT), jnp.int32),
            jax.ShapeDtypeStruct((2, NRT, 1, 2 * RT), jnp.float32),
            jax.ShapeDtypeStruct((1, 128), jnp.int32),
            jax.ShapeDtypeStruct((1, 128), jnp.int32),
        ],
        scratch_shapes=[
            pltpu.VMEM((E, 128), jnp.float32),
            pltpu.VMEM((E, 128), jnp.float32),
        ],
    )(xf, WgP, bgP, WnP, bnP, epsP)


def _scatter_body(x_hbm, d0_hbm, d1_hbm, gx_hbm,
                  r0_v, r1_v, d0_v, d1_v, sem_x0, sem_x1):
    wid = lax.axis_index("s") * 2 + lax.axis_index("c")
    base = wid * TPW
    pltpu.sync_copy(d0_hbm.at[pl.ds(base, TPW)], d0_v)
    pltpu.sync_copy(d1_hbm.at[pl.ds(base, TPW)], d1_v)

    bufs = (r0_v, r1_v)
    sems = (sem_x0, sem_x1)
    nchk = TPW // SC_C
    fired_x = []
    for c in range(nchk):
        buf = bufs[c % 2]
        if c >= 2:
            for h in fired_x[c - 2]:
                h.wait()
        pltpu.sync_copy(x_hbm.at[pl.ds(base + c * SC_C, SC_C)], buf)
        sl = pl.ds(c * SC_C, SC_C)
        i0 = d0_v[sl]
        i1 = d1_v[sl]
        fired_x.append((pltpu.async_copy(buf, gx_hbm.at[i0], sems[c % 2]),
                        pltpu.async_copy(buf, gx_hbm.at[i1], sems[c % 2])))
    for pair in fired_x[-2:]:
        for h in pair:
            h.wait()


def _dispatch_call(xf, dest0, dest1):
    mesh = plsc.VectorSubcoreMesh(core_axis_name="c", subcore_axis_name="s")
    return pl.kernel(
        _scatter_body,
        out_type=jax.ShapeDtypeStruct((GROWS, D), jnp.float32),
        mesh=mesh,
        scratch_types=[
            pltpu.VMEM((SC_C, D), jnp.float32),
            pltpu.VMEM((SC_C, D), jnp.float32),
            pltpu.VMEM((TPW,), jnp.int32),
            pltpu.VMEM((TPW,), jnp.int32),
            pltpu.SemaphoreType.DMA,
            pltpu.SemaphoreType.DMA,
        ],
    )(xf, dest0, dest1)


def _ffn_body(texp_ref, tval_ref, gx_ref, w1_ref, b1_ref, w2_ref, b2_ref,
              y_ref):
    i = pl.program_id(0)

    @pl.when(tval_ref[i] == 1)
    def _():
        xb = gx_ref[...].astype(jnp.bfloat16)
        hbs = []
        for jh in range(NHT):
            w1b = w1_ref[0, :, jh * HT:(jh + 1) * HT]
            h = jnp.dot(xb, w1b, preferred_element_type=jnp.float32)
            hbs.append(jnp.maximum(h + b1_ref[0][:, jh * HT:(jh + 1) * HT],
                                   0.0).astype(jnp.bfloat16))
        acc = b2_ref[0].astype(jnp.float32) * jnp.ones((M, 1), jnp.float32)
        for jh in range(NHT):
            w2b = w2_ref[0, jh * HT:(jh + 1) * HT, :]
            acc = acc + jnp.dot(hbs[jh], w2b, preferred_element_type=jnp.float32)
        y_ref[...] = acc


def _ffn_call(texp, tval, gxb, W1b, b1, W2b, b2):
    grid_spec = pltpu.PrefetchScalarGridSpec(
        num_scalar_prefetch=2,
        grid=(T,),
        in_specs=[
            pl.BlockSpec((M, D), lambda i, te, tv: (i, 0)),
            pl.BlockSpec((1, D, H), lambda i, te, tv: (te[i], 0, 0)),
            pl.BlockSpec((1, 1, H), lambda i, te, tv: (te[i], 0, 0)),
            pl.BlockSpec((1, H, D), lambda i, te, tv: (te[i], 0, 0)),
            pl.BlockSpec((1, 1, D), lambda i, te, tv: (te[i], 0, 0)),
        ],
        out_specs=pl.BlockSpec((M, D), lambda i, te, tv: (i, 0)),
    )
    return pl.pallas_call(
        _ffn_body,
        grid_spec=grid_spec,
        out_shape=jax.ShapeDtypeStruct((GROWS, D), jnp.float32),
    )(texp, tval, gxb, W1b, b1.reshape(E, 1, H), W2b, b2.reshape(E, 1, D))


def _combine_body(y_hbm, d0_hbm, d1_hbm, w0_hbm, w1_hbm, o_hbm,
                  a0_v, a1_v, b0_v, b1_v, o_v, d0_v, d1_v, w0_v, w1_v,
                  sem0, sem1):
    wid = lax.axis_index("s") * 2 + lax.axis_index("c")
    base = wid * TPW
    pltpu.sync_copy(d0_hbm.at[pl.ds(base, TPW)], d0_v)
    pltpu.sync_copy(d1_hbm.at[pl.ds(base, TPW)], d1_v)
    pltpu.sync_copy(w0_hbm.at[pl.ds(base, TPW)], w0_v)
    pltpu.sync_copy(w1_hbm.at[pl.ds(base, TPW)], w1_v)

    abufs = (a0_v, a1_v)
    bbufs = (b0_v, b1_v)
    sems = (sem0, sem1)
    nchk = TPW // CC

    def fire(c):
        sl = pl.ds(c * CC, CC)
        return (pltpu.async_copy(y_hbm.at[d0_v[sl]], abufs[c % 2], sems[c % 2]),
                pltpu.async_copy(y_hbm.at[d1_v[sl]], bbufs[c % 2], sems[c % 2]))

    pend = fire(0)
    for c in range(nchk):
        nxt = fire(c + 1) if c + 1 < nchk else None
        for h in pend:
            h.wait()
        a_v = abufs[c % 2]
        b_v = bbufs[c % 2]
        wsl = pl.ds(c * CC, CC)
        w0all = w0_v[wsl]
        w1all = w1_v[wsl]

        def tok(i, c2, a_v=a_v, b_v=b_v, w0all=w0all, w1all=w1all):
            lanes = jnp.full((16,), i, jnp.int32)
            wa = jnp.take_along_axis(w0all, lanes, axis=0)
            wb = jnp.take_along_axis(w1all, lanes, axis=0)
            for jj in range(D // 16):
                sl = pl.ds(jj * 16, 16)
                o_v[i, sl] = wa * a_v[i, sl] + wb * b_v[i, sl]
            return c2

        lax.fori_loop(0, CC, tok, 0)
        pltpu.sync_copy(o_v, o_hbm.at[pl.ds(base + c * CC, CC)])
        pend = nxt


def _combine_call(y, dest0, dest1, w0, w1):
    mesh = plsc.VectorSubcoreMesh(core_axis_name="c", subcore_axis_name="s")
    return pl.kernel(
        _combine_body,
        out_type=jax.ShapeDtypeStruct((TOKENS, D), jnp.float32),
        mesh=mesh,
        scratch_types=[
            pltpu.VMEM((CC, D), jnp.float32),
            pltpu.VMEM((CC, D), jnp.float32),
            pltpu.VMEM((CC, D), jnp.float32),
            pltpu.VMEM((CC, D), jnp.float32),
            pltpu.VMEM((CC, D), jnp.float32),
            pltpu.VMEM((TPW,), jnp.int32),
            pltpu.VMEM((TPW,), jnp.int32),
            pltpu.VMEM((TPW,), jnp.float32),
            pltpu.VMEM((TPW,), jnp.float32),
            pltpu.SemaphoreType.DMA,
            pltpu.SemaphoreType.DMA,
        ],
    )(y, dest0, dest1, w0, w1)


def kernel(x, Wg, bg, Wn, bn, W1, b1, W2, b2):
    B, S, _ = x.shape
    xf = x.reshape(TOKENS, D)
    WgP = jnp.zeros((D, 128), jnp.float32).at[:, :E].set(Wg)
    WnP = jnp.zeros((D, 128), jnp.float32).at[:, :E].set(Wn)
    bgP = jnp.zeros((1, 128), jnp.float32).at[0, :E].set(bg)
    bnP = jnp.zeros((1, 128), jnp.float32).at[0, :E].set(bn)

    dest, w, texp, tval = _router_call(xf, WgP, bgP, WnP, bnP, _EPSP)
    dest, w = dest[1], w[1]
    d = dest.reshape(NRT, 2, RT)
    dest0 = d[:, 0, :].reshape(TOKENS)
    dest1 = d[:, 1, :].reshape(TOKENS)
    wv = w.reshape(NRT, 2, RT)
    w0 = wv[:, 0, :].reshape(TOKENS)
    w1 = wv[:, 1, :].reshape(TOKENS)

    gx = _dispatch_call(xf, dest0, dest1)
    y = _ffn_call(texp[0, :T], tval[0, :T], gx, W1.astype(jnp.bfloat16), b1,
                  W2.astype(jnp.bfloat16), b2)
    out = _combine_call(y, dest0, dest1, w0, w1)
    return out.reshape(B, S, D)
